# baseline (speedup 1.0000x reference)
import functools
import jax
import jax.numpy as jnp
from jax import lax
from jax.experimental import pallas as pl
from jax.experimental.pallas import tpu as pltpu
from jax.experimental.pallas import tpu_sc as plsc

E = 8
K = 2
H = 768
S = 8192
FF = 4 * H

EP = 128
TS = 1024
T = 1024
FT = 512
A = S * K
NT = A // T + E
PMAX = NT * T
NF = FF // FT


def _router_body(x_ref, wr_ref, brp_ref, idx_ref, wgt_ref, bal_ref, acc_p, acc_c):
    i = pl.program_id(0)
    nprog = pl.num_programs(0)
    x = x_ref[...]
    s = jnp.dot(x, wr_ref[...], preferred_element_type=jnp.float32) + brp_ref[...]
    m = jnp.max(s, axis=-1, keepdims=True)
    ex = jnp.exp(s - m)
    probs = ex / jnp.sum(ex, axis=-1, keepdims=True)
    lanes = lax.broadcasted_iota(jnp.int32, probs.shape, 1)
    p1 = jnp.max(probs, axis=-1, keepdims=True)
    i1 = jnp.min(jnp.where(probs == p1, lanes, jnp.int32(1 << 30)), axis=-1,
                 keepdims=True)
    probs2 = jnp.where(lanes == i1, jnp.float32(-1.0), probs)
    p2 = jnp.max(probs2, axis=-1, keepdims=True)
    i2 = jnp.min(jnp.where(probs2 == p2, lanes, jnp.int32(1 << 30)), axis=-1,
                 keepdims=True)
    wsum = p1 + p2
    c = lax.broadcasted_iota(jnp.int32, (x.shape[0], 8), 1)
    idx_ref[...] = jnp.where(c == 0, i1, jnp.where(c == 1, i2, 0))
    wgt_ref[...] = jnp.where(c == 0, p1 / wsum,
                             jnp.where(c == 1, p2 / wsum, 0.0))

    @pl.when(i == 0)
    def _():
        acc_p[...] = jnp.zeros_like(acc_p)
        acc_c[...] = jnp.zeros_like(acc_c)

    acc_p[...] += jnp.sum(probs, axis=0, keepdims=True)
    acc_c[...] += jnp.sum((lanes == i1).astype(jnp.float32), axis=0,
                          keepdims=True)

    @pl.when(i == nprog - 1)
    def _():
        bal_ref[...] = jnp.full(
            (1, 1), 0.001 / (S * S), jnp.float32) * jnp.sum(
                acc_p[...] * acc_c[...], keepdims=True).reshape(1, 1)


def _router(x2d, Wr, br):
    wr_pad = jnp.zeros((H, EP), jnp.float32).at[:, :E].set(Wr)
    brp = jnp.full((1, EP), -1e30, jnp.float32).at[0, :E].set(br)
    idx, wgt, bal = pl.pallas_call(
        _router_body,
        grid=(S // TS,),
        in_specs=[
            pl.BlockSpec((TS, H), lambda i: (i, 0)),
            pl.BlockSpec((H, EP), lambda i: (0, 0)),
            pl.BlockSpec((1, EP), lambda i: (0, 0)),
        ],
        out_specs=[
            pl.BlockSpec((TS, 8), lambda i: (i, 0)),
            pl.BlockSpec((TS, 8), lambda i: (i, 0)),
            pl.BlockSpec((1, 1), lambda i: (0, 0)),
        ],
        out_shape=[
            jax.ShapeDtypeStruct((S, 8), jnp.int32),
            jax.ShapeDtypeStruct((S, 8), jnp.float32),
            jax.ShapeDtypeStruct((1, 1), jnp.float32),
        ],
        scratch_shapes=[
            pltpu.VMEM((1, EP), jnp.float32),
            pltpu.VMEM((1, EP), jnp.float32),
        ],
        compiler_params=pltpu.CompilerParams(
            dimension_semantics=("arbitrary",)),
    )(x2d, wr_pad, brp)
    return idx[:, :K], wgt[:, :K], bal[0, 0]


_NC = 2
_NW = 32
_TW = S // _NW
_CB = 64


def _dispatch_sc(x2d, pos0, pos1):
    mesh = plsc.VectorSubcoreMesh(core_axis_name="c", subcore_axis_name="s")

    @functools.partial(
        pl.kernel, mesh=mesh,
        out_type=jax.ShapeDtypeStruct((PMAX, H), jnp.float32),
        scratch_types=[
            pltpu.VMEM((_CB, H), jnp.float32),
            pltpu.VMEM((_CB,), jnp.int32),
            pltpu.VMEM((_CB,), jnp.int32),
            pltpu.SemaphoreType.DMA,
        ],
    )
    def k(x_hbm, p0_hbm, p1_hbm, xs_hbm, rows_v, i0_v, i1_v, sem):
        wid = lax.axis_index("s") * _NC + lax.axis_index("c")

        def body(j, carry):
            base = wid * _TW + j * _CB
            pltpu.sync_copy(x_hbm.at[pl.ds(base, _CB)], rows_v)
            pltpu.sync_copy(p0_hbm.at[pl.ds(base, _CB)], i0_v)
            pltpu.sync_copy(p1_hbm.at[pl.ds(base, _CB)], i1_v)
            cp0 = pltpu.async_copy(rows_v, xs_hbm.at[i0_v], sem)
            cp1 = pltpu.async_copy(rows_v, xs_hbm.at[i1_v], sem)
            cp0.wait()
            cp1.wait()
            return carry

        lax.fori_loop(0, _TW // _CB, body, 0)

    return k(x2d, pos0, pos1)


def _mm_body(te_ref, xs_ref, w1_ref, b1_ref, w2_ref, b2_ref, out_ref):
    x = xs_ref[...].astype(jnp.bfloat16)
    acc = b2_ref[0] + jnp.zeros((T, H), jnp.float32)
    for f in range(NF):
        h = jnp.dot(x, w1_ref[0, :, f * FT:(f + 1) * FT].astype(jnp.bfloat16),
                    preferred_element_type=jnp.float32)
        h = h + b1_ref[0, :, f * FT:(f + 1) * FT]
        a = jnp.maximum(h, 0.0)
        a = a * a
        acc = acc + jnp.dot(a.astype(jnp.bfloat16),
                            w2_ref[0, f * FT:(f + 1) * FT, :].astype(jnp.bfloat16),
                            preferred_element_type=jnp.float32)
    out_ref[...] = acc


def _grouped_mm(xs, W1, b1, W2, b2, tile_expert):
    grid_spec = pltpu.PrefetchScalarGridSpec(
        num_scalar_prefetch=1,
        grid=(NT,),
        in_specs=[
            pl.BlockSpec((T, H), lambda t, te: (t, 0)),
            pl.BlockSpec((1, H, FF), lambda t, te: (te[t], 0, 0)),
            pl.BlockSpec((1, 1, FF), lambda t, te: (te[t], 0, 0)),
            pl.BlockSpec((1, FF, H), lambda t, te: (te[t], 0, 0)),
            pl.BlockSpec((1, 1, H), lambda t, te: (te[t], 0, 0)),
        ],
        out_specs=pl.BlockSpec((T, H), lambda t, te: (t, 0)),
    )
    return pl.pallas_call(
        _mm_body,
        grid_spec=grid_spec,
        out_shape=jax.ShapeDtypeStruct((PMAX, H), jnp.float32),
        compiler_params=pltpu.CompilerParams(
            dimension_semantics=("arbitrary",)),
    )(tile_expert, xs, W1, b1.reshape(E, 1, FF), W2, b2.reshape(E, 1, H))


def kernel(x, Wr, br, W1, b1, W2, b2):
    x2d = x.reshape(S, H)
    eid, w, bal = _router(x2d, Wr, br)

    ef = eid.reshape(-1)
    oh = (ef[:, None] == jnp.arange(E, dtype=jnp.int32)[None, :])
    csum = jnp.cumsum(oh.astype(jnp.int32), axis=0)
    rank = jnp.take_along_axis(csum, ef[:, None], axis=1)[:, 0] - 1
    counts = csum[-1]
    pc = ((counts + T - 1) // T) * T
    base = jnp.concatenate([jnp.zeros((1,), jnp.int32),
                            jnp.cumsum(pc)[:-1].astype(jnp.int32)])
    pos = base[ef] + rank
    tb = base // T
    t = jnp.arange(NT, dtype=jnp.int32)
    tile_expert = jnp.sum((t[:, None] >= tb[None, :]).astype(jnp.int32),
                          axis=1) - 1

    pos2 = pos.reshape(S, K)
    xs = _dispatch_sc(x2d, pos2[:, 0], pos2[:, 1])
    ys = _grouped_mm(xs, W1, b1, W2, b2, tile_expert)
    out = w[:, 0:1] * ys[pos2[:, 0]] + w[:, 1:2] * ys[pos2[:, 1]]
    return out.reshape(1, S, H), bal


# speedup vs baseline: 1.0588x; 1.0588x over previous
import functools
import jax
import jax.numpy as jnp
from jax import lax
from jax.experimental import pallas as pl
from jax.experimental.pallas import tpu as pltpu
from jax.experimental.pallas import tpu_sc as plsc

E = 8
K = 2
H = 768
S = 8192
FF = 4 * H

EP = 128
TS = 1024
T = 512
FT = 768
A = S * K
NT = A // T + E
PMAX = NT * T
NF = FF // FT


def _router_body(x_ref, wr_ref, brp_ref, idx_ref, wgt_ref, bal_ref, acc_p, acc_c):
    i = pl.program_id(0)
    nprog = pl.num_programs(0)
    x = x_ref[...]
    s = jnp.dot(x, wr_ref[...], preferred_element_type=jnp.float32) + brp_ref[...]
    m = jnp.max(s, axis=-1, keepdims=True)
    ex = jnp.exp(s - m)
    probs = ex / jnp.sum(ex, axis=-1, keepdims=True)
    lanes = lax.broadcasted_iota(jnp.int32, probs.shape, 1)
    p1 = jnp.max(probs, axis=-1, keepdims=True)
    i1 = jnp.min(jnp.where(probs == p1, lanes, jnp.int32(1 << 30)), axis=-1,
                 keepdims=True)
    probs2 = jnp.where(lanes == i1, jnp.float32(-1.0), probs)
    p2 = jnp.max(probs2, axis=-1, keepdims=True)
    i2 = jnp.min(jnp.where(probs2 == p2, lanes, jnp.int32(1 << 30)), axis=-1,
                 keepdims=True)
    wsum = p1 + p2
    c = lax.broadcasted_iota(jnp.int32, (x.shape[0], 8), 1)
    idx_ref[...] = jnp.where(c == 0, i1, jnp.where(c == 1, i2, 0))
    wgt_ref[...] = jnp.where(c == 0, p1 / wsum,
                             jnp.where(c == 1, p2 / wsum, 0.0))

    @pl.when(i == 0)
    def _():
        acc_p[...] = jnp.zeros_like(acc_p)
        acc_c[...] = jnp.zeros_like(acc_c)

    acc_p[...] += jnp.sum(probs, axis=0, keepdims=True)
    acc_c[...] += jnp.sum((lanes == i1).astype(jnp.float32), axis=0,
                          keepdims=True)

    @pl.when(i == nprog - 1)
    def _():
        bal_ref[...] = jnp.full(
            (1, 1), 0.001 / (S * S), jnp.float32) * jnp.sum(
                acc_p[...] * acc_c[...], keepdims=True).reshape(1, 1)


def _router(x2d, Wr, br):
    wr_pad = jnp.zeros((H, EP), jnp.float32).at[:, :E].set(Wr)
    brp = jnp.full((1, EP), -1e30, jnp.float32).at[0, :E].set(br)
    idx, wgt, bal = pl.pallas_call(
        _router_body,
        grid=(S // TS,),
        in_specs=[
            pl.BlockSpec((TS, H), lambda i: (i, 0)),
            pl.BlockSpec((H, EP), lambda i: (0, 0)),
            pl.BlockSpec((1, EP), lambda i: (0, 0)),
        ],
        out_specs=[
            pl.BlockSpec((TS, 8), lambda i: (i, 0)),
            pl.BlockSpec((TS, 8), lambda i: (i, 0)),
            pl.BlockSpec((1, 1), lambda i: (0, 0)),
        ],
        out_shape=[
            jax.ShapeDtypeStruct((S, 8), jnp.int32),
            jax.ShapeDtypeStruct((S, 8), jnp.float32),
            jax.ShapeDtypeStruct((1, 1), jnp.float32),
        ],
        scratch_shapes=[
            pltpu.VMEM((1, EP), jnp.float32),
            pltpu.VMEM((1, EP), jnp.float32),
        ],
        compiler_params=pltpu.CompilerParams(
            dimension_semantics=("arbitrary",)),
    )(x2d, wr_pad, brp)
    return idx[:, :K], wgt[:, :K], bal[0, 0]


_NC = 2
_NW = 32
_TW = S // _NW
_CB = 64


def _dispatch_sc(x2d, pos0, pos1):
    mesh = plsc.VectorSubcoreMesh(core_axis_name="c", subcore_axis_name="s")

    @functools.partial(
        pl.kernel, mesh=mesh,
        out_type=jax.ShapeDtypeStruct((PMAX, H), jnp.float32),
        scratch_types=[
            pltpu.VMEM((_CB, H), jnp.float32),
            pltpu.VMEM((_CB,), jnp.int32),
            pltpu.VMEM((_CB,), jnp.int32),
            pltpu.SemaphoreType.DMA,
        ],
    )
    def k(x_hbm, p0_hbm, p1_hbm, xs_hbm, rows_v, i0_v, i1_v, sem):
        wid = lax.axis_index("s") * _NC + lax.axis_index("c")

        def body(j, carry):
            base = wid * _TW + j * _CB
            pltpu.sync_copy(x_hbm.at[pl.ds(base, _CB)], rows_v)
            pltpu.sync_copy(p0_hbm.at[pl.ds(base, _CB)], i0_v)
            pltpu.sync_copy(p1_hbm.at[pl.ds(base, _CB)], i1_v)
            cp0 = pltpu.async_copy(rows_v, xs_hbm.at[i0_v], sem)
            cp1 = pltpu.async_copy(rows_v, xs_hbm.at[i1_v], sem)
            cp0.wait()
            cp1.wait()
            return carry

        lax.fori_loop(0, _TW // _CB, body, 0)

    return k(x2d, pos0, pos1)


def _mm_body(te_ref, xs_ref, w1_ref, b1_ref, w2_ref, b2_ref, out_ref):
    x = xs_ref[...].astype(jnp.bfloat16)
    acc = b2_ref[0] + jnp.zeros((T, H), jnp.float32)
    for f in range(NF):
        h = jnp.dot(x, w1_ref[0, :, f * FT:(f + 1) * FT].astype(jnp.bfloat16),
                    preferred_element_type=jnp.float32)
        h = h + b1_ref[0, :, f * FT:(f + 1) * FT]
        a = jnp.maximum(h, 0.0)
        a = a * a
        acc = acc + jnp.dot(a.astype(jnp.bfloat16),
                            w2_ref[0, f * FT:(f + 1) * FT, :].astype(jnp.bfloat16),
                            preferred_element_type=jnp.float32)
    out_ref[...] = acc


def _grouped_mm(xs, W1, b1, W2, b2, tile_expert):
    grid_spec = pltpu.PrefetchScalarGridSpec(
        num_scalar_prefetch=1,
        grid=(NT,),
        in_specs=[
            pl.BlockSpec((T, H), lambda t, te: (t, 0)),
            pl.BlockSpec((1, H, FF), lambda t, te: (te[t], 0, 0)),
            pl.BlockSpec((1, 1, FF), lambda t, te: (te[t], 0, 0)),
            pl.BlockSpec((1, FF, H), lambda t, te: (te[t], 0, 0)),
            pl.BlockSpec((1, 1, H), lambda t, te: (te[t], 0, 0)),
        ],
        out_specs=pl.BlockSpec((T, H), lambda t, te: (t, 0)),
    )
    return pl.pallas_call(
        _mm_body,
        grid_spec=grid_spec,
        out_shape=jax.ShapeDtypeStruct((PMAX, H), jnp.float32),
        compiler_params=pltpu.CompilerParams(
            dimension_semantics=("arbitrary",)),
    )(tile_expert, xs, W1, b1.reshape(E, 1, FF), W2, b2.reshape(E, 1, H))


def kernel(x, Wr, br, W1, b1, W2, b2):
    x2d = x.reshape(S, H)
    eid, w, bal = _router(x2d, Wr, br)

    ef = eid.reshape(-1)
    oh = (ef[:, None] == jnp.arange(E, dtype=jnp.int32)[None, :])
    csum = jnp.cumsum(oh.astype(jnp.int32), axis=0)
    rank = jnp.take_along_axis(csum, ef[:, None], axis=1)[:, 0] - 1
    counts = csum[-1]
    pc = ((counts + T - 1) // T) * T
    base = jnp.concatenate([jnp.zeros((1,), jnp.int32),
                            jnp.cumsum(pc)[:-1].astype(jnp.int32)])
    pos = base[ef] + rank
    tb = base // T
    t = jnp.arange(NT, dtype=jnp.int32)
    tile_expert = jnp.sum((t[:, None] >= tb[None, :]).astype(jnp.int32),
                          axis=1) - 1

    pos2 = pos.reshape(S, K)
    xs = _dispatch_sc(x2d, pos2[:, 0], pos2[:, 1])
    ys = _grouped_mm(xs, W1, b1, W2, b2, tile_expert)
    out = w[:, 0:1] * ys[pos2[:, 0]] + w[:, 1:2] * ys[pos2[:, 1]]
    return out.reshape(1, S, H), bal
